# baseline (speedup 1.0000x reference)
import functools

import jax
import jax.numpy as jnp
from jax import lax
from jax.experimental import pallas as pl
from jax.experimental.pallas import tpu as pltpu
from jax.experimental.pallas import tpu_sc as plsc

_LANES = 16


def _build(n: int, n_workers: int):
    per = n // n_workers
    assert per % _LANES == 0 and per % 8 == 0
    chunks = per // _LANES
    mesh = plsc.VectorSubcoreMesh(core_axis_name="c", subcore_axis_name="s")

    @functools.partial(
        pl.kernel,
        out_type=jax.ShapeDtypeStruct((n,), jnp.float32),
        mesh=mesh,
        scratch_types=[
            pltpu.VMEM((_LANES,), jnp.float32),
            pltpu.VMEM((per,), jnp.int32),
            pltpu.VMEM((per,), jnp.float32),
        ],
    )
    def run(x_hbm, w_hbm, out_hbm, tab, xv, ov):
        wid = lax.axis_index("s") * 2 + lax.axis_index("c")
        base = wid * per
        pltpu.sync_copy(w_hbm, tab)
        t = jnp.exp(tab[...])
        pltpu.sync_copy(x_hbm.at[pl.ds(base, per)], xv)

        def body(i, carry):
            idx = xv[pl.ds(i * _LANES, _LANES)]
            ov[pl.ds(i * _LANES, _LANES)] = t.at[idx].get(
                mode="promise_in_bounds"
            )
            return carry

        lax.fori_loop(0, chunks, body, 0, unroll=8)
        pltpu.sync_copy(ov, out_hbm.at[pl.ds(base, per)])

    return run


def kernel(x, weight):
    xf = x.reshape(-1).astype(jnp.int32)
    wpad = jnp.pad(weight.astype(jnp.float32), (0, _LANES - weight.shape[0]))
    return _build(xf.shape[0], 32)(xf, wpad)


# speedup vs baseline: 1.0820x; 1.0820x over previous
import functools

import jax
import jax.numpy as jnp
from jax import lax
from jax.experimental import pallas as pl
from jax.experimental.pallas import tpu as pltpu
from jax.experimental.pallas import tpu_sc as plsc

_LANES = 16


def _build(n: int, n_workers: int):
    per = n // n_workers
    assert per % _LANES == 0 and per % 8 == 0
    chunks = per // _LANES
    mesh = plsc.VectorSubcoreMesh(core_axis_name="c", subcore_axis_name="s")

    @functools.partial(
        pl.kernel,
        out_type=jax.ShapeDtypeStruct((n,), jnp.float32),
        mesh=mesh,
        scratch_types=[
            pltpu.VMEM((_LANES,), jnp.float32),
            pltpu.VMEM((per,), jnp.int32),
            pltpu.VMEM((per,), jnp.float32),
        ],
    )
    def run(x_hbm, w_hbm, out_hbm, tab, xv, ov):
        wid = lax.axis_index("s") * 2 + lax.axis_index("c")
        base = wid * per
        pltpu.sync_copy(w_hbm, tab)
        t = jnp.exp(tab[...])
        pltpu.sync_copy(x_hbm.at[pl.ds(base, per)], xv)

        @plsc.parallel_loop(0, chunks, 1, unroll=8)
        def body(i):
            idx = xv[pl.ds(i * _LANES, _LANES)]
            ov[pl.ds(i * _LANES, _LANES)] = t.at[idx].get(
                mode="promise_in_bounds"
            )
        pltpu.sync_copy(ov, out_hbm.at[pl.ds(base, per)])

    return run


def kernel(x, weight):
    xf = x.reshape(-1).astype(jnp.int32)
    wpad = jnp.pad(weight.astype(jnp.float32), (0, _LANES - weight.shape[0]))
    return _build(xf.shape[0], 32)(xf, wpad)


# speedup vs baseline: 1.3539x; 1.2513x over previous
import functools

import jax
import jax.numpy as jnp
from jax import lax
from jax.experimental import pallas as pl
from jax.experimental.pallas import tpu as pltpu
from jax.experimental.pallas import tpu_sc as plsc

_LANES = 16


def _build(n_rows: int, n_cols: int, n_workers: int):
    rows_per = n_rows // n_workers
    per = rows_per * n_cols
    assert rows_per * n_workers == n_rows
    tail = n_cols - _LANES
    assert 0 < tail <= _LANES and per % 8 == 0
    mesh = plsc.VectorSubcoreMesh(core_axis_name="c", subcore_axis_name="s")

    @functools.partial(
        pl.kernel,
        out_type=jax.ShapeDtypeStruct((n_rows * n_cols,), jnp.float32),
        mesh=mesh,
        scratch_types=[
            pltpu.VMEM((_LANES,), jnp.float32),
            pltpu.VMEM((rows_per, n_cols), jnp.int32),
            pltpu.VMEM((per,), jnp.float32),
        ],
    )
    def run(x_hbm, w_hbm, out_hbm, tab, xv, ov):
        wid = lax.axis_index("s") * 2 + lax.axis_index("c")
        row0 = wid * rows_per
        pltpu.sync_copy(w_hbm, tab)
        t = jnp.exp(tab[...])
        pltpu.sync_copy(x_hbm.at[pl.ds(row0, rows_per), :], xv)

        @plsc.parallel_loop(0, rows_per, 1, unroll=8)
        def body(r):
            ia = xv[r, pl.ds(0, _LANES)]
            ib = xv[r, pl.ds(tail, _LANES)]
            ov[pl.ds(r * n_cols, _LANES)] = t.at[ia].get(
                mode="promise_in_bounds")
            ov[pl.ds(r * n_cols + tail, _LANES)] = t.at[ib].get(
                mode="promise_in_bounds")

        pltpu.sync_copy(ov, out_hbm.at[pl.ds(row0 * n_cols, per)])

    return run


def kernel(x, weight):
    n_rows, n_cols = x.shape
    wpad = jnp.pad(weight.astype(jnp.float32), (0, _LANES - weight.shape[0]))
    return _build(n_rows, n_cols, 32)(x.astype(jnp.int32), wpad)
